# baseline (speedup 1.0000x reference)
import jax
import jax.numpy as jnp
from jax import lax
from jax.experimental import pallas as pl
from jax.experimental.pallas import tpu as pltpu

DIM = 256
H = W = 36
HW = H * W
POOL = 4
SCALE = 20.0
EPS_BN = 1e-5


def _shift_rows(a, o, rows, cols):
    if o > 0:
        return jnp.concatenate(
            [a[o:], jnp.zeros((o, cols), jnp.float32)], axis=0)
    if o < 0:
        return jnp.concatenate(
            [jnp.zeros((-o, cols), jnp.float32), a[:rows + o]], axis=0)
    return a


def _finish_conv(z, b_ref, y_ref, p_ref):
    wcol = lax.rem(lax.broadcasted_iota(jnp.int32, (HW, 1), 0), W)
    acc = (z[1]
           + jnp.where(wcol == 0, 0.0, _shift_rows(z[0], -1, HW, DIM))
           + jnp.where(wcol == W - 1, 0.0, _shift_rows(z[2], 1, HW, DIM))
           + b_ref[...])
    y_ref[0] = acc
    p_ref[0] = jnp.concatenate(
        [jnp.sum(acc, axis=0, keepdims=True),
         jnp.sum(acc * acc, axis=0, keepdims=True),
         jnp.zeros((6, DIM), jnp.float32)], axis=0)


def _make_conv_kernel(act):
    def body(x_ref, w_ref, b_ref, *rest):
        if act:
            s_ref, t_ref, y_ref, p_ref = rest
        else:
            y_ref, p_ref = rest
        x = x_ref[0]
        if act:
            x = jnp.maximum(x * s_ref[...] + t_ref[...], 0.0)
        xcat = jnp.concatenate(
            [_shift_rows(x, -W, HW, DIM), x, _shift_rows(x, W, HW, DIM)],
            axis=1)
        z = [jnp.dot(xcat, w_ref[d], preferred_element_type=jnp.float32)
             for d in range(3)]
        _finish_conv(z, b_ref, y_ref, p_ref)
    return body


def _conv_cf_kernel(x_ref, w_ref, b_ref, y_ref, p_ref):
    xt = x_ref[0]
    lo = jnp.concatenate(
        [jnp.zeros((DIM, W), jnp.float32), xt[:, :HW - W]], axis=1)
    hi = jnp.concatenate(
        [xt[:, W:], jnp.zeros((DIM, W), jnp.float32)], axis=1)
    xcat = jnp.concatenate([lo, xt, hi], axis=0)
    z = [lax.dot_general(xcat, w_ref[d],
                         dimension_numbers=(((0,), (0,)), ((), ())),
                         preferred_element_type=jnp.float32)
         for d in range(3)]
    _finish_conv(z, b_ref, y_ref, p_ref)


def _conv_bn(x, taps, b, s=None, t=None, cf=False):
    act = s is not None
    if cf:
        n = x.shape[0]
        kern = _conv_cf_kernel
        inputs = [x, taps, b.reshape(1, DIM)]
        in_specs = [
            pl.BlockSpec((1, DIM, HW), lambda i: (i, 0, 0)),
            pl.BlockSpec((3, 3 * DIM, DIM), lambda i: (0, 0, 0)),
            pl.BlockSpec((1, DIM), lambda i: (0, 0)),
        ]
    else:
        n = x.shape[0]
        kern = _make_conv_kernel(act)
        inputs = [x, taps, b.reshape(1, DIM)]
        in_specs = [
            pl.BlockSpec((1, HW, DIM), lambda i: (i, 0, 0)),
            pl.BlockSpec((3, 3 * DIM, DIM), lambda i: (0, 0, 0)),
            pl.BlockSpec((1, DIM), lambda i: (0, 0)),
        ]
        if act:
            inputs += [s.reshape(1, DIM), t.reshape(1, DIM)]
            in_specs += [pl.BlockSpec((1, DIM), lambda i: (0, 0)),
                         pl.BlockSpec((1, DIM), lambda i: (0, 0))]
    return pl.pallas_call(
        kern,
        grid=(n,),
        in_specs=in_specs,
        out_specs=[pl.BlockSpec((1, HW, DIM), lambda i: (i, 0, 0)),
                   pl.BlockSpec((1, 8, DIM), lambda i: (i, 0, 0))],
        out_shape=[jax.ShapeDtypeStruct((n, HW, DIM), jnp.float32),
                   jax.ShapeDtypeStruct((n, 8, DIM), jnp.float32)],
        compiler_params=pltpu.CompilerParams(
            dimension_semantics=("parallel",)),
        name="conv_bn",
    )(*inputs)


def _bn_affine(p, g, beta, n):
    cnt = jnp.float32(n * HW)
    m = jnp.sum(p[:, 0, :], axis=0) / cnt
    ex2 = jnp.sum(p[:, 1, :], axis=0) / cnt
    v = ex2 - m * m
    s = g / jnp.sqrt(v + EPS_BN)
    return s, beta - m * s


def _corner_weights(coord, lo_max):
    k = jnp.clip(jnp.floor(coord), 0.0, lo_max)
    sfrac = coord - k
    p1 = 0.5 * sfrac * sfrac
    p0 = sfrac - p1
    k2 = k[:, :, None]
    p02 = p0[:, :, None]
    p12 = p1[:, :, None]
    npp = coord.shape[0]
    wio = lax.broadcasted_iota(
        jnp.int32, (npp, POOL + 1, W), 2).astype(jnp.float32)
    tri = jnp.where(
        k2 > 0,
        jnp.where(wio < k2, jnp.where(wio == 0, 0.5, 1.0),
                  jnp.where(wio == k2, 0.5, 0.0)),
        0.0)
    return (tri + jnp.where(wio == k2, p02, 0.0)
            + jnp.where(wio == k2 + 1, p12, 0.0))


def _make_pool_kernel(npp):
    def body(f_ref, prop_ref, s_ref, t_ref, o_ref):
        fmap = jnp.maximum(f_ref[0] * s_ref[...] + t_ref[...], 0.0)
        p = prop_ref[0]
        x1 = p[:, 0:1] * SCALE
        y1 = p[:, 1:2] * SCALE
        bw = p[:, 2:3] * (SCALE / POOL)
        bh = p[:, 3:4] * (SCALE / POOL)
        gridv = lax.broadcasted_iota(
            jnp.int32, (1, POOL + 1), 1).astype(jnp.float32)
        xs = jnp.clip(x1 + bw * gridv, 0.0, W - 1.0)
        ys = jnp.clip(y1 + bh * gridv, 0.0, H - 1.0)
        hx = _corner_weights(xs, W - 2.0)
        hy = _corner_weights(ys, H - 2.0)
        cx = hx[:, 1:] - hx[:, :-1]
        cy = hy[:, 1:] - hy[:, :-1]
        area = bw * bh
        inva = jnp.where(area > 1e-8, 1.0 / jnp.maximum(area, 1e-8), 0.0)
        cx = cx * inva[:, :, None]
        nrij = npp * POOL * POOL
        cx2 = jnp.broadcast_to(cx[:, :, None, :],
                               (npp, POOL, POOL, W)).reshape(nrij, W)
        cy2 = jnp.broadcast_to(cy[:, None, :, :],
                               (npp, POOL, POOL, H)).reshape(nrij, H)
        lane = lax.broadcasted_iota(jnp.int32, (W, HW), 1)
        row = lax.broadcasted_iota(jnp.int32, (W, HW), 0)
        tile_m = (lax.rem(lane, W) == row).astype(jnp.float32)
        stretch_m = (lax.div(lane, W) == row).astype(jnp.float32)
        wt = (jnp.dot(cx2, tile_m, preferred_element_type=jnp.float32)
              * jnp.dot(cy2, stretch_m, preferred_element_type=jnp.float32))
        pooled = jnp.dot(wt, fmap, preferred_element_type=jnp.float32)
        o_ref[0] = pooled.reshape(npp, POOL * POOL * DIM)
    return body


def _pool(fmap, props, s3, t3):
    n, npp = props.shape[0], props.shape[1]
    return pl.pallas_call(
        _make_pool_kernel(npp),
        grid=(n,),
        in_specs=[
            pl.BlockSpec((1, HW, DIM), lambda i: (i, 0, 0)),
            pl.BlockSpec((1, npp, 4), lambda i: (i, 0, 0)),
            pl.BlockSpec((1, DIM), lambda i: (0, 0)),
            pl.BlockSpec((1, DIM), lambda i: (0, 0)),
        ],
        out_specs=pl.BlockSpec((1, npp, POOL * POOL * DIM),
                               lambda i: (i, 0, 0)),
        out_shape=jax.ShapeDtypeStruct((n, npp, POOL * POOL * DIM),
                                       jnp.float32),
        compiler_params=pltpu.CompilerParams(
            dimension_semantics=("parallel",)),
        name="prroi_pool",
    )(fmap, props, s3.reshape(1, DIM), t3.reshape(1, DIM))


def _make_fc_kernel(nrois):
    def body(q_ref, w_ref, b_ref, g_ref, beta_ref, iw_ref, ib_ref, o_ref):
        fcx = jnp.dot(q_ref[...], w_ref[...],
                      preferred_element_type=jnp.float32) + b_ref[...]
        m = jnp.mean(fcx, axis=0, keepdims=True)
        v = jnp.mean(fcx * fcx, axis=0, keepdims=True) - m * m
        x = (fcx - m) / jnp.sqrt(v + EPS_BN) * g_ref[...] + beta_ref[...]
        x = jnp.maximum(x, 0.0)
        o_ref[...] = jnp.dot(x, iw_ref[...],
                             preferred_element_type=jnp.float32) + ib_ref[...]
    return body


def _fc_head(q, fcw, fcb, g, beta, iw, ib):
    nrois = q.shape[0]
    return pl.pallas_call(
        _make_fc_kernel(nrois),
        out_shape=jax.ShapeDtypeStruct((nrois, 1), jnp.float32),
        name="fc_iou_head",
    )(q, fcw, fcb.reshape(1, DIM), g.reshape(1, DIM), beta.reshape(1, DIM),
      iw, ib.reshape(1, 1))


def kernel(feat, proposals, conv1_w, conv1_b, bn1_g, bn1_b, conv2_w, conv2_b,
           bn2_g, bn2_b, conv3_w, conv3_b, bn3_g, bn3_b, fc_w, fc_b,
           fcbn_g, fcbn_b, iou_w, iou_b):
    ni, ns, npp = proposals.shape[0], proposals.shape[1], proposals.shape[2]
    n = ni * ns

    x0 = feat.reshape(n, DIM, HW)
    taps1 = conv1_w.transpose(3, 2, 1, 0).reshape(3, 3 * DIM, DIM)
    taps2 = conv2_w.transpose(3, 2, 1, 0).reshape(3, 3 * DIM, DIM)
    taps3 = conv3_w.transpose(3, 2, 1, 0).reshape(3, 3 * DIM, DIM)

    y1, p1 = _conv_bn(x0, taps1, conv1_b, cf=True)
    s1, t1 = _bn_affine(p1, bn1_g, bn1_b, n)
    y2, p2 = _conv_bn(y1, taps2, conv2_b, s1, t1)
    s2, t2 = _bn_affine(p2, bn2_g, bn2_b, n)
    y3, p3 = _conv_bn(y2, taps3, conv3_b, s2, t2)
    s3, t3 = _bn_affine(p3, bn3_g, bn3_b, n)

    q = _pool(y3, proposals.reshape(n, npp, 4), s3, t3)

    fcw = fc_w.reshape(DIM, DIM, POOL, POOL).transpose(3, 2, 1, 0)
    fcw = fcw.reshape(POOL * POOL * DIM, DIM)
    iou = _fc_head(q.reshape(n * npp, POOL * POOL * DIM), fcw, fc_b,
                   fcbn_g, fcbn_b, iou_w.T, iou_b)
    return iou.reshape(ni, ns, npp)


# speedup vs baseline: 1.2067x; 1.2067x over previous
import jax
import jax.numpy as jnp
from jax import lax
from jax.experimental import pallas as pl
from jax.experimental.pallas import tpu as pltpu

DIM = 256
H = W = 36
HW = H * W
POOL = 4
SCALE = 20.0
EPS_BN = 1e-5


def _shift_rows(a, o, rows, cols):
    if o > 0:
        return jnp.concatenate(
            [a[o:], jnp.zeros((o, cols), jnp.float32)], axis=0)
    if o < 0:
        return jnp.concatenate(
            [jnp.zeros((-o, cols), jnp.float32), a[:rows + o]], axis=0)
    return a


def _finish_conv(z, b_ref, y_ref, p_ref):
    wcol = lax.rem(lax.broadcasted_iota(jnp.int32, (HW, 1), 0), W)
    acc = (z[1]
           + jnp.where(wcol == 0, 0.0, _shift_rows(z[0], -1, HW, DIM))
           + jnp.where(wcol == W - 1, 0.0, _shift_rows(z[2], 1, HW, DIM))
           + b_ref[...])
    y_ref[...] = acc
    p_ref[...] = jnp.concatenate(
        [jnp.sum(acc, axis=0, keepdims=True),
         jnp.sum(acc * acc, axis=0, keepdims=True),
         jnp.zeros((6, DIM), jnp.float32)], axis=0)


def _make_conv_kernel(act, bm):
    def body(x_ref, w_ref, b_ref, *rest):
        if act:
            s_ref, t_ref, y_ref, p_ref = rest
        else:
            y_ref, p_ref = rest
        for im in range(bm):
            x = x_ref[im]
            if act:
                x = jnp.maximum(x * s_ref[...] + t_ref[...], 0.0)
            xcat = jnp.concatenate(
                [_shift_rows(x, -W, HW, DIM), x, _shift_rows(x, W, HW, DIM)],
                axis=1)
            z = [jnp.dot(xcat, w_ref[d], preferred_element_type=jnp.float32)
                 for d in range(3)]
            _finish_conv(z, b_ref, y_ref.at[im], p_ref.at[im])
    return body


def _make_conv_cf_kernel(bm):
    def body(x_ref, w_ref, b_ref, y_ref, p_ref):
        for im in range(bm):
            xt = x_ref[im * DIM:(im + 1) * DIM].reshape(DIM, HW)
            lo = jnp.concatenate(
                [jnp.zeros((DIM, W), jnp.float32), xt[:, :HW - W]], axis=1)
            hi = jnp.concatenate(
                [xt[:, W:], jnp.zeros((DIM, W), jnp.float32)], axis=1)
            xcat = jnp.concatenate([lo, xt, hi], axis=0)
            z = [lax.dot_general(xcat, w_ref[d],
                                 dimension_numbers=(((0,), (0,)), ((), ())),
                                 preferred_element_type=jnp.float32)
                 for d in range(3)]
            _finish_conv(z, b_ref, y_ref.at[im], p_ref.at[im])
    return body


def _conv_bn(x, taps, b, s=None, t=None, cf=False, bm=2):
    act = s is not None
    inputs = [x, taps, b.reshape(1, DIM)]
    if cf:
        n = x.shape[0] // DIM
        kern = _make_conv_cf_kernel(bm)
        in_specs = [
            pl.BlockSpec((bm * DIM, H, W), lambda i: (i, 0, 0)),
            pl.BlockSpec((3, 3 * DIM, DIM), lambda i: (0, 0, 0)),
            pl.BlockSpec((1, DIM), lambda i: (0, 0)),
        ]
    else:
        n = x.shape[0]
        kern = _make_conv_kernel(act, bm)
        in_specs = [
            pl.BlockSpec((bm, HW, DIM), lambda i: (i, 0, 0)),
            pl.BlockSpec((3, 3 * DIM, DIM), lambda i: (0, 0, 0)),
            pl.BlockSpec((1, DIM), lambda i: (0, 0)),
        ]
        if act:
            inputs += [s.reshape(1, DIM), t.reshape(1, DIM)]
            in_specs += [pl.BlockSpec((1, DIM), lambda i: (0, 0)),
                         pl.BlockSpec((1, DIM), lambda i: (0, 0))]
    return pl.pallas_call(
        kern,
        grid=(n // bm,),
        in_specs=in_specs,
        out_specs=[pl.BlockSpec((bm, HW, DIM), lambda i: (i, 0, 0)),
                   pl.BlockSpec((bm, 8, DIM), lambda i: (i, 0, 0))],
        out_shape=[jax.ShapeDtypeStruct((n, HW, DIM), jnp.float32),
                   jax.ShapeDtypeStruct((n, 8, DIM), jnp.float32)],
        compiler_params=pltpu.CompilerParams(
            dimension_semantics=("parallel",)),
        name="conv_bn",
    )(*inputs)


def _bn_affine(p, g, beta, n):
    cnt = jnp.float32(n * HW)
    m = jnp.sum(p[:, 0, :], axis=0) / cnt
    ex2 = jnp.sum(p[:, 1, :], axis=0) / cnt
    v = ex2 - m * m
    s = g / jnp.sqrt(v + EPS_BN)
    return s, beta - m * s


def _corner_weights(coord, lo_max):
    k = jnp.clip(jnp.floor(coord), 0.0, lo_max)
    sfrac = coord - k
    p1 = 0.5 * sfrac * sfrac
    p0 = sfrac - p1
    k2 = k[:, :, None]
    p02 = p0[:, :, None]
    p12 = p1[:, :, None]
    npp = coord.shape[0]
    wio = lax.broadcasted_iota(
        jnp.int32, (npp, POOL + 1, W), 2).astype(jnp.float32)
    tri = jnp.where(
        k2 > 0,
        jnp.where(wio < k2, jnp.where(wio == 0, 0.5, 1.0),
                  jnp.where(wio == k2, 0.5, 0.0)),
        0.0)
    return (tri + jnp.where(wio == k2, p02, 0.0)
            + jnp.where(wio == k2 + 1, p12, 0.0))


def _make_pool_kernel(npp):
    def body(f_ref, prop_ref, s_ref, t_ref, o_ref):
        bm = prop_ref.shape[0]
        nr = bm * npp
        p = prop_ref[...].reshape(nr, 4)
        x1 = p[:, 0:1] * SCALE
        y1 = p[:, 1:2] * SCALE
        bw = p[:, 2:3] * (SCALE / POOL)
        bh = p[:, 3:4] * (SCALE / POOL)
        gridv = lax.broadcasted_iota(
            jnp.int32, (1, POOL + 1), 1).astype(jnp.float32)
        xs = jnp.clip(x1 + bw * gridv, 0.0, W - 1.0)
        ys = jnp.clip(y1 + bh * gridv, 0.0, H - 1.0)
        hx = _corner_weights(xs, W - 2.0)
        hy = _corner_weights(ys, H - 2.0)
        cx = hx[:, 1:] - hx[:, :-1]
        cy = hy[:, 1:] - hy[:, :-1]
        area = bw * bh
        inva = jnp.where(area > 1e-8, 1.0 / jnp.maximum(area, 1e-8), 0.0)
        cx = cx * inva[:, :, None]
        nrij = npp * POOL * POOL
        cx2 = jnp.broadcast_to(cx[:, :, None, :],
                               (nr, POOL, POOL, W)).reshape(bm * nrij, W)
        cy2 = jnp.broadcast_to(cy[:, None, :, :],
                               (nr, POOL, POOL, H)).reshape(bm * nrij, H)
        lane = lax.broadcasted_iota(jnp.int32, (W, HW), 1)
        row = lax.broadcasted_iota(jnp.int32, (W, HW), 0)
        tile_m = (lax.rem(lane, W) == row).astype(jnp.float32)
        stretch_m = (lax.div(lane, W) == row).astype(jnp.float32)
        wt = (jnp.dot(cx2, tile_m, preferred_element_type=jnp.float32)
              * jnp.dot(cy2, stretch_m, preferred_element_type=jnp.float32))
        for im in range(bm):
            fmap = jnp.maximum(
                f_ref[im] * s_ref[...] + t_ref[...], 0.0)
            pooled = jnp.dot(wt[im * nrij:(im + 1) * nrij], fmap,
                             preferred_element_type=jnp.float32)
            o_ref[im] = pooled.reshape(npp, POOL * POOL * DIM)
    return body


def _pool(fmap, props, s3, t3, bm=2):
    n, npp = props.shape[0], props.shape[1]
    return pl.pallas_call(
        _make_pool_kernel(npp),
        grid=(n // bm,),
        in_specs=[
            pl.BlockSpec((bm, HW, DIM), lambda i: (i, 0, 0)),
            pl.BlockSpec((bm, npp, 4), lambda i: (i, 0, 0)),
            pl.BlockSpec((1, DIM), lambda i: (0, 0)),
            pl.BlockSpec((1, DIM), lambda i: (0, 0)),
        ],
        out_specs=pl.BlockSpec((bm, npp, POOL * POOL * DIM),
                               lambda i: (i, 0, 0)),
        out_shape=jax.ShapeDtypeStruct((n, npp, POOL * POOL * DIM),
                                       jnp.float32),
        compiler_params=pltpu.CompilerParams(
            dimension_semantics=("parallel",)),
        name="prroi_pool",
    )(fmap, props, s3.reshape(1, DIM), t3.reshape(1, DIM))


def _make_fc_kernel(nrois):
    def body(q_ref, w_ref, b_ref, g_ref, beta_ref, iw_ref, ib_ref, o_ref):
        fcx = jnp.dot(q_ref[...], w_ref[...],
                      preferred_element_type=jnp.float32) + b_ref[...]
        m = jnp.mean(fcx, axis=0, keepdims=True)
        v = jnp.mean(fcx * fcx, axis=0, keepdims=True) - m * m
        x = (fcx - m) / jnp.sqrt(v + EPS_BN) * g_ref[...] + beta_ref[...]
        x = jnp.maximum(x, 0.0)
        o_ref[...] = jnp.dot(x, iw_ref[...],
                             preferred_element_type=jnp.float32) + ib_ref[...]
    return body


def _fc_head(q, fcw, fcb, g, beta, iw, ib):
    nrois = q.shape[0]
    return pl.pallas_call(
        _make_fc_kernel(nrois),
        out_shape=jax.ShapeDtypeStruct((nrois, 1), jnp.float32),
        name="fc_iou_head",
    )(q, fcw, fcb.reshape(1, DIM), g.reshape(1, DIM), beta.reshape(1, DIM),
      iw, ib.reshape(1, 1))


def kernel(feat, proposals, conv1_w, conv1_b, bn1_g, bn1_b, conv2_w, conv2_b,
           bn2_g, bn2_b, conv3_w, conv3_b, bn3_g, bn3_b, fc_w, fc_b,
           fcbn_g, fcbn_b, iou_w, iou_b):
    ni, ns, npp = proposals.shape[0], proposals.shape[1], proposals.shape[2]
    n = ni * ns

    x0 = feat.reshape(n * DIM, H, W)
    taps1 = conv1_w.transpose(3, 2, 1, 0).reshape(3, 3 * DIM, DIM)
    taps2 = conv2_w.transpose(3, 2, 1, 0).reshape(3, 3 * DIM, DIM)
    taps3 = conv3_w.transpose(3, 2, 1, 0).reshape(3, 3 * DIM, DIM)

    y1, p1 = _conv_bn(x0, taps1, conv1_b, cf=True)
    s1, t1 = _bn_affine(p1, bn1_g, bn1_b, n)
    y2, p2 = _conv_bn(y1, taps2, conv2_b, s1, t1)
    s2, t2 = _bn_affine(p2, bn2_g, bn2_b, n)
    y3, p3 = _conv_bn(y2, taps3, conv3_b, s2, t2)
    s3, t3 = _bn_affine(p3, bn3_g, bn3_b, n)

    q = _pool(y3, proposals.reshape(n, npp, 4), s3, t3)

    fcw = fc_w.reshape(DIM, DIM, POOL, POOL).transpose(3, 2, 1, 0)
    fcw = fcw.reshape(POOL * POOL * DIM, DIM)
    iou = _fc_head(q.reshape(n * npp, POOL * POOL * DIM), fcw, fc_b,
                   fcbn_g, fcbn_b, iou_w.T, iou_b)
    return iou.reshape(ni, ns, npp)
